# baseline (speedup 1.0000x reference)
import jax
import jax.numpy as jnp
from jax.experimental import pallas as pl
from jax.experimental.pallas import tpu as pltpu

_D_MODEL = 128
_HALF = 64
_ROWS_PER_BLOCK = 49152
_TILES_PER_BLOCK = _ROWS_PER_BLOCK // 128

_PIO2 = 1.5707963267948966
_S1 = -1.621898383e-1
_C1 = -4.997966614e-1
_C2 = 4.053367444e-2


def _pos_enc_kernel(pos_ref, wk2_ref, g_ref, out_ref):
    pos = pos_ref[...].reshape(1, _ROWS_PER_BLOCK)
    wcol2 = jnp.broadcast_to(wk2_ref[...], (_HALF, _D_MODEL))
    g = g_ref[...]

    for c in range(_TILES_PER_BLOCK):
        p = pos[:, c * 128:(c + 1) * 128]
        t = wcol2 * p
        q = jnp.round(t).astype(jnp.int32)
        d = t - q.astype(jnp.float32)

        r = (d * _PIO2).astype(jnp.bfloat16)
        q16 = q.astype(jnp.int16)
        t16 = t.astype(jnp.bfloat16)

        z = r * r
        sin_r = r + r * (z * jnp.bfloat16(_S1))
        cos_r = jnp.bfloat16(1.0) + z * (jnp.bfloat16(_C1) + z * jnp.bfloat16(_C2))

        qb = pltpu.bitcast(q16, jnp.int32)
        ab = qb & 0x00010001
        sgn0 = (qb & 0x00020002) << 14
        sgn1 = sgn0 ^ (ab << 15)
        m = pltpu.bitcast(ab, jnp.int16) == 0
        v0 = pltpu.bitcast(
            pltpu.bitcast(jnp.where(m, sin_r, cos_r), jnp.int32) ^ sgn0,
            jnp.bfloat16)
        v1 = pltpu.bitcast(
            pltpu.bitcast(jnp.where(m, cos_r, sin_r), jnp.int32) ^ sgn1,
            jnp.bfloat16)
        v1 = jnp.where(t16 != 0, v1, jnp.bfloat16(0.0))

        m_ = jnp.concatenate([v0, v1], axis=0)
        out_ref[c * 128:(c + 1) * 128, :] = jax.lax.dot_general(
            m_, g, (((0,), (1,)), ((), ())),
            preferred_element_type=jnp.float32)


def kernel(positions, w_k):
    n = positions.shape[0]
    d = w_k.shape[0]
    num_blocks = pl.cdiv(n, _ROWS_PER_BLOCK)
    wk2u = w_k[0::2] * jnp.float32(2.0 / jnp.pi)
    lane = jnp.arange(d, dtype=jnp.int32)
    gsel = jax.nn.one_hot((lane >> 1) + ((lane & 1) << 6), d, dtype=jnp.bfloat16)
    return pl.pallas_call(
        _pos_enc_kernel,
        grid=(num_blocks,),
        in_specs=[
            pl.BlockSpec((_ROWS_PER_BLOCK,), lambda i: (i,)),
            pl.BlockSpec((_HALF, 1), lambda i: (0, 0)),
            pl.BlockSpec((d, d), lambda i: (0, 0)),
        ],
        out_specs=pl.BlockSpec((_ROWS_PER_BLOCK, d), lambda i: (i, 0)),
        out_shape=jax.ShapeDtypeStruct((n, d), jnp.float32),
        compiler_params=pltpu.CompilerParams(
            dimension_semantics=("parallel",),
        ),
    )(positions, wk2u.reshape(_HALF, 1), gsel)


# speedup vs baseline: 1.0304x; 1.0304x over previous
import jax
import jax.numpy as jnp
from jax.experimental import pallas as pl
from jax.experimental.pallas import tpu as pltpu

_D_MODEL = 128
_HALF = 64
_ROWS_PER_BLOCK = 32768
_TILES_PER_BLOCK = _ROWS_PER_BLOCK // 128

_PIO2 = 1.5707963267948966
_S1 = -1.621898383e-1
_C1 = -4.997966614e-1
_C2 = 4.053367444e-2


def _pos_enc_kernel(pos_ref, wk2_ref, g_ref, out_ref):
    pos = pos_ref[...].reshape(1, _ROWS_PER_BLOCK)
    wcol2 = jnp.broadcast_to(wk2_ref[...], (_HALF, _D_MODEL))
    g = g_ref[...]

    for c in range(_TILES_PER_BLOCK):
        p = pos[:, c * 128:(c + 1) * 128]
        t = wcol2 * p
        q = jnp.round(t).astype(jnp.int32)
        d = t - q.astype(jnp.float32)

        r = (d * _PIO2).astype(jnp.bfloat16)
        q16 = q.astype(jnp.int16)
        t16 = t.astype(jnp.bfloat16)

        z = r * r
        sin_r = r + r * (z * jnp.bfloat16(_S1))
        cos_r = jnp.bfloat16(1.0) + z * (jnp.bfloat16(_C1) + z * jnp.bfloat16(_C2))

        qb = pltpu.bitcast(q16, jnp.int32)
        ab = qb & 0x00010001
        sgn0 = (qb & 0x00020002) << 14
        sgn1 = sgn0 ^ (ab << 15)
        m = pltpu.bitcast(ab, jnp.int16) == 0
        v0 = pltpu.bitcast(
            pltpu.bitcast(jnp.where(m, sin_r, cos_r), jnp.int32) ^ sgn0,
            jnp.bfloat16)
        v1 = pltpu.bitcast(
            pltpu.bitcast(jnp.where(m, cos_r, sin_r), jnp.int32) ^ sgn1,
            jnp.bfloat16)
        v1 = jnp.where(t16 != 0, v1, jnp.bfloat16(0.0))

        m_ = jnp.concatenate([v0, v1], axis=0)
        out_ref[c * 128:(c + 1) * 128, :] = jax.lax.dot_general(
            m_, g, (((0,), (1,)), ((), ())),
            preferred_element_type=jnp.float32)


def kernel(positions, w_k):
    n = positions.shape[0]
    d = w_k.shape[0]
    num_blocks = pl.cdiv(n, _ROWS_PER_BLOCK)
    wk2u = w_k[0::2] * jnp.float32(2.0 / jnp.pi)
    lane = jnp.arange(d, dtype=jnp.int32)
    gsel = jax.nn.one_hot((lane >> 1) + ((lane & 1) << 6), d, dtype=jnp.bfloat16)
    return pl.pallas_call(
        _pos_enc_kernel,
        grid=(num_blocks,),
        in_specs=[
            pl.BlockSpec((_ROWS_PER_BLOCK,), lambda i: (i,)),
            pl.BlockSpec((_HALF, 1), lambda i: (0, 0)),
            pl.BlockSpec((d, d), lambda i: (0, 0)),
        ],
        out_specs=pl.BlockSpec((_ROWS_PER_BLOCK, d), lambda i: (i, 0)),
        out_shape=jax.ShapeDtypeStruct((n, d), jnp.float32),
        compiler_params=pltpu.CompilerParams(
            dimension_semantics=("parallel",),
        ),
    )(positions, wk2u.reshape(_HALF, 1), gsel)
